# baseline (speedup 1.0000x reference)
import functools

import jax
import jax.numpy as jnp
from jax import lax
from jax.experimental import pallas as pl
from jax.experimental.pallas import tpu as pltpu
from jax.experimental.pallas import tpu_sc as plsc

S, N, E = 4096, 4, 1024
B = S * N
NW = 32
RPW = B // NW
CH = 32
NCHUNK = RPW // CH
LANES = 16
NGRP = E // LANES
GPB = 16
NCB = NGRP // GPB
NBUF = 3


def _make_sc_call():
    mesh = plsc.VectorSubcoreMesh(core_axis_name="c", subcore_axis_name="s")

    @functools.partial(
        pl.kernel,
        mesh=mesh,
        out_type=jax.ShapeDtypeStruct((S, N, E), jnp.float32),
        scratch_types=[
            *[pltpu.VMEM((CH // N, N, E), jnp.float32)
              for _ in range(NBUF)],
            pltpu.VMEM((2, E), jnp.float32),
            pltpu.VMEM((RPW,), jnp.int32),
            pltpu.VMEM((RPW * LANES,), jnp.float32),
            pltpu.SemaphoreType.DMA((NBUF,)),
            pltpu.SemaphoreType.DMA((NBUF,)),
        ],
    )
    def sc_call(seq_hbm, tok_hbm, table_hbm, out_hbm, *rest):
        bufs = rest[:NBUF]
        table_v, tok_v, f_v, in_sem, out_sem = rest[NBUF:]
        wid = lax.axis_index("s") * 2 + lax.axis_index("c")
        base = wid * RPW
        sbase = wid * (RPW // N)
        SCH = CH // N

        for b in range(1):
            pltpu.async_copy(
                seq_hbm.at[pl.ds(sbase + b * SCH, SCH)], bufs[b],
                in_sem.at[b])

        pltpu.sync_copy(tok_hbm.at[pl.ds(base, RPW)], tok_v)
        pltpu.sync_copy(table_hbm, table_v)

        dnums = lax.GatherDimensionNumbers(
            offset_dims=(), collapsed_slice_dims=(0,), start_index_map=(0,))

        def bcast_body(g, _):
            tv = tok_v[pl.ds(g * LANES, LANES)].astype(jnp.float32)
            for l in range(LANES):
                idx = jnp.full((LANES, 1), l, jnp.int32)
                f_v[pl.ds((g * LANES + l) * LANES, LANES)] = lax.gather(
                    tv, idx, dnums, slice_sizes=(1,),
                    mode=lax.GatherScatterMode.PROMISE_IN_BOUNDS)
            return 0

        lax.fori_loop(0, RPW // LANES, bcast_body, 0)

        def wait_in(b):
            pltpu.make_async_copy(
                seq_hbm.at[pl.ds(0, SCH)], bufs[b], in_sem.at[b]).wait()

        def wait_out(b):
            pltpu.make_async_copy(
                bufs[b], out_hbm.at[pl.ds(0, SCH)], out_sem.at[b]).wait()

        for i in range(NCHUNK):
            b = i % NBUF
            row0 = i * CH
            wait_in(b)
            buf = bufs[b]

            j = i + 1
            bj = (b + 1) % NBUF
            if j < NCHUNK:
                if j >= NBUF:
                    wait_out(bj)
                pltpu.async_copy(
                    seq_hbm.at[pl.ds(sbase + j * SCH, SCH)], bufs[bj],
                    in_sem.at[bj])

            for cb in range(NCB):
                o0 = cb * GPB * LANES
                t0s = [table_v[0, pl.ds(o0 + k * LANES, LANES)]
                       for k in range(GPB)]
                dcs = [table_v[1, pl.ds(o0 + k * LANES, LANES)] - t0s[k]
                       for k in range(GPB)]

                def row_body(r, _, buf=buf, row0=row0, o0=o0,
                             t0s=t0s, dcs=dcs):
                    fr = f_v[pl.ds((row0 + r) * LANES, LANES)]
                    a = r // N
                    n = r % N
                    for k in range(GPB):
                        o = o0 + k * LANES
                        s = buf[a, n, pl.ds(o, LANES)]
                        buf[a, n, pl.ds(o, LANES)] = (
                            s + t0s[k] + fr * dcs[k])
                    return 0

                lax.fori_loop(0, CH, row_body, 0)

            pltpu.async_copy(
                buf, out_hbm.at[pl.ds(sbase + i * SCH, SCH)],
                out_sem.at[b])

        for b in range(NBUF):
            wait_out(b)

    return sc_call


_sc_call = _make_sc_call()


@jax.jit
def kernel(seq_input, token_type_input, token_type_embeddings):
    tok = token_type_input.reshape(B).astype(jnp.int32)
    return _sc_call(seq_input, tok, token_type_embeddings)


# speedup vs baseline: 1.1385x; 1.1385x over previous
import functools

import jax
import jax.numpy as jnp
from jax import lax
from jax.experimental import pallas as pl
from jax.experimental.pallas import tpu as pltpu
from jax.experimental.pallas import tpu_sc as plsc

S, N, E = 4096, 4, 1024
B = S * N
NW = 32
RPW = B // NW
CH = 16
NCHUNK = RPW // CH
LANES = 16
NGRP = E // LANES
GPB = 16
NCB = NGRP // GPB
NBUF = 4


def _make_sc_call():
    mesh = plsc.VectorSubcoreMesh(core_axis_name="c", subcore_axis_name="s")

    @functools.partial(
        pl.kernel,
        mesh=mesh,
        out_type=jax.ShapeDtypeStruct((S, N, E), jnp.float32),
        scratch_types=[
            *[pltpu.VMEM((CH // N, N, E), jnp.float32)
              for _ in range(NBUF)],
            pltpu.VMEM((2, E), jnp.float32),
            pltpu.VMEM((RPW,), jnp.int32),
            pltpu.VMEM((RPW * LANES,), jnp.float32),
            pltpu.SemaphoreType.DMA((NBUF,)),
            pltpu.SemaphoreType.DMA((NBUF,)),
        ],
    )
    def sc_call(seq_hbm, tok_hbm, table_hbm, out_hbm, *rest):
        bufs = rest[:NBUF]
        table_v, tok_v, f_v, in_sem, out_sem = rest[NBUF:]
        wid = lax.axis_index("s") * 2 + lax.axis_index("c")
        base = wid * RPW
        sbase = wid * (RPW // N)
        SCH = CH // N

        for b in range(NBUF - 2):
            pltpu.async_copy(
                seq_hbm.at[pl.ds(sbase + b * SCH, SCH)], bufs[b],
                in_sem.at[b])

        pltpu.sync_copy(tok_hbm.at[pl.ds(base, RPW)], tok_v)
        pltpu.sync_copy(table_hbm, table_v)

        dnums = lax.GatherDimensionNumbers(
            offset_dims=(), collapsed_slice_dims=(0,), start_index_map=(0,))

        def bcast_body(g, _):
            tv = tok_v[pl.ds(g * LANES, LANES)].astype(jnp.float32)
            for l in range(LANES):
                idx = jnp.full((LANES, 1), l, jnp.int32)
                f_v[pl.ds((g * LANES + l) * LANES, LANES)] = lax.gather(
                    tv, idx, dnums, slice_sizes=(1,),
                    mode=lax.GatherScatterMode.PROMISE_IN_BOUNDS)
            return 0

        lax.fori_loop(0, RPW // LANES, bcast_body, 0)

        def wait_in(b):
            pltpu.make_async_copy(
                seq_hbm.at[pl.ds(0, SCH)], bufs[b], in_sem.at[b]).wait()

        def wait_out(b):
            pltpu.make_async_copy(
                bufs[b], out_hbm.at[pl.ds(0, SCH)], out_sem.at[b]).wait()

        def super_step(g, _):
            for b in range(NBUF):
                i = g * NBUF + b
                row0 = i * CH
                wait_in(b)
                buf = bufs[b]

                j = i + 2
                bj = (b + 2) % NBUF

                @pl.when(jnp.logical_and(j >= NBUF, j < NCHUNK))
                def _():
                    wait_out(bj)

                @pl.when(j < NCHUNK)
                def _():
                    pltpu.async_copy(
                        seq_hbm.at[pl.ds(sbase + j * SCH, SCH)], bufs[bj],
                        in_sem.at[bj])

                for cb in range(NCB):
                    o0 = cb * GPB * LANES
                    t0s = [table_v[0, pl.ds(o0 + k * LANES, LANES)]
                           for k in range(GPB)]
                    t1s = [table_v[1, pl.ds(o0 + k * LANES, LANES)]
                           for k in range(GPB)]

                    def row_body(r, _, buf=buf, row0=row0, o0=o0,
                                 t0s=t0s, t1s=t1s):
                        fr = f_v[pl.ds((row0 + r) * LANES, LANES)]
                        m = fr > 0.5
                        a = r // N
                        n = r % N
                        for k in range(GPB):
                            o = o0 + k * LANES
                            s = buf[a, n, pl.ds(o, LANES)]
                            buf[a, n, pl.ds(o, LANES)] = (
                                s + jnp.where(m, t1s[k], t0s[k]))
                        return 0

                    lax.fori_loop(0, CH, row_body, 0)

                pltpu.async_copy(
                    buf, out_hbm.at[pl.ds(sbase + i * SCH, SCH)],
                    out_sem.at[b])

            return 0

        lax.fori_loop(0, NCHUNK // NBUF, super_step, 0)

        for b in range(NBUF):
            wait_out(b)

    return sc_call


_sc_call = _make_sc_call()


@jax.jit
def kernel(seq_input, token_type_input, token_type_embeddings):
    tok = token_type_input.reshape(B).astype(jnp.int32)
    return _sc_call(seq_input, tok, token_type_embeddings)
